# Initial kernel scaffold; baseline (speedup 1.0000x reference)
import jax
import jax.numpy as jnp
from jax.experimental import pallas as pl


def kernel(x, edge_index, global_ctx, theta_t, params):
    raise NotImplementedError("write your pallas kernel here")


# speedup vs baseline: 9.4084x; 9.4084x over previous
import functools

import jax
import jax.numpy as jnp
from jax import lax
from jax.experimental import pallas as pl
from jax.experimental.pallas import tpu as pltpu
from jax.experimental.pallas import tpu_sc as plsc

_CHUNK = 128
_BLK = 1000


@functools.partial(jax.jit, static_argnums=(2, 3))
def _sc_degree(dst, zeros_n, N, E):
  n_chunks = E // _CHUNK
  mesh = plsc.VectorSubcoreMesh(core_axis_name="c", subcore_axis_name="s")
  blk = 16 * 128
  Np = ((N + blk - 1) // blk) * blk
  stripe = Np // 16

  @functools.partial(
      pl.kernel,
      out_type=jax.ShapeDtypeStruct((2, Np), jnp.float32),
      mesh=mesh,
      scratch_types=[
          pltpu.VMEM((_CHUNK,), jnp.int32),
          pltpu.VMEM((_CHUNK,), jnp.float32),
          pltpu.VMEM_SHARED((Np,), jnp.float32),
      ],
  )
  def k(dst_hbm, zeros_hbm, out_hbm, dst_v, ones_v, acc):
    c = lax.axis_index("c")
    s = lax.axis_index("s")
    w = s * 2 + c
    for i in range(_CHUNK // 16):
      ones_v[pl.ds(i * 16, 16)] = jnp.ones((16,), jnp.float32)
    @pl.when(s == 0)
    def _():
      pltpu.sync_copy(zeros_hbm, acc)
    plsc.subcore_barrier()
    base_trips = n_chunks // 32
    extra = n_chunks - base_trips * 32
    trips = base_trips + jnp.where(w < extra, 1, 0).astype(jnp.int32)

    def body(j, carry):
      chunk = j * 32 + w
      base = pl.multiple_of(chunk * _CHUNK, _CHUNK)
      pltpu.sync_copy(dst_hbm.at[pl.ds(base, _CHUNK)], dst_v)
      pltpu.sync_copy(ones_v, acc.at[dst_v], add=True)
      return carry

    lax.fori_loop(0, trips, body, 0)
    plsc.subcore_barrier()

    def flush(ci):
      @pl.when(c == ci)
      def _():
        off = pl.multiple_of(s * stripe, 128)
        pltpu.sync_copy(acc.at[pl.ds(off, stripe)],
                        out_hbm.at[ci].at[pl.ds(off, stripe)])

    flush(0)
    flush(1)

  return k(dst, zeros_n)


@functools.partial(jax.jit, static_argnums=(4, 5))
def _sc_aggregate(hws, src, dst, zeros_n32, N, E):
  n_chunks = E // _CHUNK
  mesh = plsc.VectorSubcoreMesh(core_axis_name="c", subcore_axis_name="s")
  Np = ((N + 127) // 128) * 128
  rpt = Np // 16

  @functools.partial(
      pl.kernel,
      out_type=jax.ShapeDtypeStruct((2, Np, 32), jnp.float32),
      mesh=mesh,
      scratch_types=[
          pltpu.VMEM((_CHUNK,), jnp.int32),
          pltpu.VMEM((_CHUNK,), jnp.int32),
          pltpu.VMEM((_CHUNK, 32), jnp.float32),
          pltpu.VMEM_SHARED((Np, 32), jnp.float32),
          pltpu.SemaphoreType.DMA,
      ],
      compiler_params=pltpu.CompilerParams(use_tc_tiling_on_sc=False),
  )
  def k(hws_hbm, src_hbm, dst_hbm, zeros_hbm, out_hbm,
        src_v, dst_v, rows_v, acc, sem):
    c = lax.axis_index("c")
    s = lax.axis_index("s")
    row0 = pl.multiple_of(s * rpt, 8)
    pltpu.sync_copy(zeros_hbm.at[pl.ds(row0, rpt)],
                    acc.at[pl.ds(row0, rpt)])
    plsc.subcore_barrier()
    base_trips = n_chunks // 16
    extra = n_chunks - base_trips * 16
    trips = base_trips + jnp.where(s < extra, 1, 0).astype(jnp.int32)

    def run(ci):
      @pl.when(c == ci)
      def _():
        def body(j, carry):
          chunk = j * 16 + s
          base = pl.multiple_of(chunk * _CHUNK, _CHUNK)
          pltpu.sync_copy(src_hbm.at[pl.ds(base, _CHUNK)], src_v)
          pltpu.sync_copy(dst_hbm.at[pl.ds(base, _CHUNK)], dst_v)
          pltpu.async_copy(hws_hbm.at[ci].at[src_v], rows_v, sem).wait()
          pltpu.sync_copy(rows_v, acc.at[dst_v], add=True)
          return carry

        lax.fori_loop(0, trips, body, 0)

    run(0)
    run(1)
    plsc.subcore_barrier()

    def flush(ci):
      @pl.when(c == ci)
      def _():
        pltpu.sync_copy(acc.at[pl.ds(row0, rpt)],
                        out_hbm.at[ci].at[pl.ds(row0, rpt)])

    flush(0)
    flush(1)

  return k(hws, src, dst, zeros_n32)


def _mm1_body(x_ref, w_ref, deg_ref, out_ref):
  dinv = lax.rsqrt(deg_ref[...])
  hws = jnp.dot(x_ref[...], w_ref[...],
                preferred_element_type=jnp.float32) * dinv
  out_ref[0, ...] = hws[:, :32]
  out_ref[1, ...] = hws[:, 32:]


def _mm1(x, w, deg2d, N, D):
  grid = (N // _BLK,)
  return pl.pallas_call(
      _mm1_body,
      grid=grid,
      in_specs=[
          pl.BlockSpec((_BLK, D), lambda i: (i, 0)),
          pl.BlockSpec((D, 64), lambda i: (0, 0)),
          pl.BlockSpec((_BLK, 1), lambda i: (i, 0)),
      ],
      out_specs=pl.BlockSpec((2, _BLK, 32), lambda i: (0, i, 0)),
      out_shape=jax.ShapeDtypeStruct((2, N, 32), jnp.float32),
  )(x, w, deg2d)


def _core_h(agg_ref, hws_ref, deg_ref, stats_ref, gamma_ref, beta_ref, n):
  dinv = lax.rsqrt(deg_ref[...])
  core = jnp.concatenate(
      [agg_ref[0] + hws_ref[0], agg_ref[1] + hws_ref[1]], axis=1) * dinv
  mu = stats_ref[0:1, :] / n
  var = stats_ref[1:2, :] / n - mu * mu
  h = gamma_ref[...] * (core - mu) * lax.rsqrt(var + 1e-5) + beta_ref[...]
  return jnp.maximum(h, 0.0)


def _stats_body(agg_ref, hws_ref, deg_ref, out_ref):
  i = pl.program_id(0)
  @pl.when(i == 0)
  def _():
    out_ref[...] = jnp.zeros_like(out_ref)
  dinv = lax.rsqrt(deg_ref[...])
  core = jnp.concatenate(
      [agg_ref[0] + hws_ref[0], agg_ref[1] + hws_ref[1]], axis=1) * dinv
  out_ref[...] += jnp.stack(
      [jnp.sum(core, axis=0), jnp.sum(core * core, axis=0)], axis=0)


def _stats(agg, hws, deg2d, N):
  grid = (N // _BLK,)
  return pl.pallas_call(
      _stats_body,
      grid=grid,
      in_specs=[
          pl.BlockSpec((2, _BLK, 32), lambda i: (0, i, 0)),
          pl.BlockSpec((2, _BLK, 32), lambda i: (0, i, 0)),
          pl.BlockSpec((_BLK, 1), lambda i: (i, 0)),
      ],
      out_specs=pl.BlockSpec((2, 64), lambda i: (0, 0)),
      out_shape=jax.ShapeDtypeStruct((2, 64), jnp.float32),
  )(agg, hws, deg2d)


def _mm23_body(agg_ref, hws_ref, deg_ref, stats_ref, gamma_ref, beta_ref,
               w_ref, out_ref, *, n):
  h = _core_h(agg_ref, hws_ref, deg_ref, stats_ref, gamma_ref, beta_ref, n)
  dinv = lax.rsqrt(deg_ref[...])
  hws = jnp.dot(h, w_ref[...], preferred_element_type=jnp.float32) * dinv
  out_ref[0, ...] = hws[:, :32]
  out_ref[1, ...] = hws[:, 32:]


def _mm23(agg, hws, deg2d, stats, gamma, beta, w, N):
  grid = (N // _BLK,)
  return pl.pallas_call(
      functools.partial(_mm23_body, n=float(N)),
      grid=grid,
      in_specs=[
          pl.BlockSpec((2, _BLK, 32), lambda i: (0, i, 0)),
          pl.BlockSpec((2, _BLK, 32), lambda i: (0, i, 0)),
          pl.BlockSpec((_BLK, 1), lambda i: (i, 0)),
          pl.BlockSpec((2, 64), lambda i: (0, 0)),
          pl.BlockSpec((1, 64), lambda i: (0, 0)),
          pl.BlockSpec((1, 64), lambda i: (0, 0)),
          pl.BlockSpec((64, 64), lambda i: (0, 0)),
      ],
      out_specs=pl.BlockSpec((2, _BLK, 32), lambda i: (0, i, 0)),
      out_shape=jax.ShapeDtypeStruct((2, N, 32), jnp.float32),
  )(agg, hws, deg2d, stats, gamma, beta, w)


def _final_body(agg_ref, hws_ref, deg_ref, stats_ref, gamma_ref, beta_ref,
                gctx_ref, w1a_ref, w1b_ref, fb1_ref, w2_ref, fb2_ref,
                dw1_ref, db1_ref, dw2_ref, db2_ref, theta_ref,
                delta_ref, thout_ref, *, n):
  h3 = _core_h(agg_ref, hws_ref, deg_ref, stats_ref, gamma_ref, beta_ref, n)
  gv = jnp.dot(gctx_ref[...], w1b_ref[...],
               preferred_element_type=jnp.float32) + fb1_ref[...]
  t = jnp.maximum(
      jnp.dot(h3, w1a_ref[...], preferred_element_type=jnp.float32) + gv, 0.0)
  hf = h3 + jnp.dot(t, w2_ref[...],
                    preferred_element_type=jnp.float32) + fb2_ref[...]
  d1 = jnp.maximum(
      jnp.dot(hf, dw1_ref[...], preferred_element_type=jnp.float32)
      + db1_ref[...], 0.0)
  dfull = jnp.dot(d1, dw2_ref[...], preferred_element_type=jnp.float32)
  delta = dfull[:, 0:1] + db2_ref[...]
  a = theta_ref[...] + delta
  delta_ref[...] = delta
  thout_ref[...] = jnp.arctan2(jnp.sin(a), jnp.cos(a))


def _final(agg, hws, deg2d, stats, gamma, beta, gctx, w1a, w1b, fb1, w2, fb2,
           dw1, db1, dw2p, db2, theta2d, N):
  grid = (N // _BLK,)
  full = lambda shape: pl.BlockSpec(shape, lambda i: tuple(0 for _ in shape))
  return pl.pallas_call(
      functools.partial(_final_body, n=float(N)),
      grid=grid,
      in_specs=[
          pl.BlockSpec((2, _BLK, 32), lambda i: (0, i, 0)),
          pl.BlockSpec((2, _BLK, 32), lambda i: (0, i, 0)),
          pl.BlockSpec((_BLK, 1), lambda i: (i, 0)),
          full((2, 64)),
          full((1, 64)),
          full((1, 64)),
          full((1, 4)),
          full((64, 64)),
          full((4, 64)),
          full((1, 64)),
          full((64, 64)),
          full((1, 64)),
          full((64, 64)),
          full((1, 64)),
          full((64, 128)),
          full((1, 1)),
          pl.BlockSpec((_BLK, 1), lambda i: (i, 0)),
      ],
      out_specs=[
          pl.BlockSpec((_BLK, 1), lambda i: (i, 0)),
          pl.BlockSpec((_BLK, 1), lambda i: (i, 0)),
      ],
      out_shape=[
          jax.ShapeDtypeStruct((N, 1), jnp.float32),
          jax.ShapeDtypeStruct((N, 1), jnp.float32),
      ],
  )(agg, hws, deg2d, stats, gamma, beta, gctx, w1a, w1b, fb1, w2, fb2,
    dw1, db1, dw2p, db2, theta2d)


def kernel(x, edge_index, global_ctx, theta_t, params):
  N, D = x.shape
  E = edge_index.shape[1]
  src = edge_index[0]
  dst = edge_index[1]
  Np = ((N + 16 * 128 - 1) // (16 * 128)) * (16 * 128)
  zeros_np = jnp.zeros((Np,), jnp.float32)
  zeros_n32 = jnp.zeros((((N + 127) // 128) * 128, 32), jnp.float32)

  parts = _sc_degree(dst, zeros_np, N, E)
  deg2d = (parts[0, :N] + parts[1, :N] + 1.0).reshape(N, 1)

  hws = _mm1(x, params["gcn_W"][0], deg2d, N, D)
  r = lambda v: v.reshape(1, -1)
  for li in range(3):
    agg = _sc_aggregate(hws, src, dst, zeros_n32, N, E)
    stats = _stats(agg, hws, deg2d, N)
    gamma = r(params["bn_gamma"][li])
    beta = r(params["bn_beta"][li])
    if li < 2:
      hws = _mm23(agg, hws, deg2d, stats, gamma, beta,
                  params["gcn_W"][li + 1], N)
    else:
      dw2p = jnp.pad(params["dec_W2"], ((0, 0), (0, 127)))
      delta2d, theta2d = _final(
          agg, hws, deg2d, stats, gamma, beta, global_ctx,
          params["fus_W1"][:64], params["fus_W1"][64:], r(params["fus_b1"]),
          params["fus_W2"], r(params["fus_b2"]),
          params["dec_W1"], r(params["dec_b1"]),
          dw2p, params["dec_b2"].reshape(1, 1),
          theta_t.reshape(N, 1), N)
  return delta2d[:, 0], theta2d[:, 0]


# speedup vs baseline: 11.1401x; 1.1841x over previous
import functools

import jax
import jax.numpy as jnp
from jax import lax
from jax.experimental import pallas as pl
from jax.experimental.pallas import tpu as pltpu
from jax.experimental.pallas import tpu_sc as plsc

_CHUNK = 128
_BLK = 1000


@functools.partial(jax.jit, static_argnums=(2, 3))
def _sc_degree(dst4, zeros_np, N, E):
  mesh = plsc.VectorSubcoreMesh(core_axis_name="c", subcore_axis_name="s")
  blk = 16 * 128
  Np = ((N + blk - 1) // blk) * blk
  stripe = Np // 16
  n_groups = dst4.shape[1] // _G

  @functools.partial(
      pl.kernel,
      out_type=jax.ShapeDtypeStruct((2, Np), jnp.float32),
      mesh=mesh,
      scratch_types=[
          pltpu.VMEM((_G, _CHUNK), jnp.int32),
          pltpu.VMEM((_CHUNK,), jnp.float32),
          pltpu.VMEM_SHARED((Np,), jnp.float32),
          pltpu.SemaphoreType.DMA,
      ],
  )
  def k(dst_hbm, zeros_hbm, out_hbm, dst_v, ones_v, acc, ssem):
    c = lax.axis_index("c")
    s = lax.axis_index("s")
    w = s * 2 + c
    for i in range(_CHUNK // 16):
      ones_v[pl.ds(i * 16, 16)] = jnp.ones((16,), jnp.float32)
    @pl.when(s == 0)
    def _():
      pltpu.sync_copy(zeros_hbm, acc)
    plsc.subcore_barrier()

    def body(g, carry):
      g0 = pl.multiple_of(g * _G, _G)
      pltpu.sync_copy(dst_hbm.at[w].at[pl.ds(g0, _G)], dst_v)
      descs = [pltpu.async_copy(ones_v, acc.at[dst_v.at[b]], ssem, add=True)
               for b in range(_G)]
      for d in descs:
        d.wait()
      return carry

    lax.fori_loop(0, n_groups, body, 0)
    plsc.subcore_barrier()

    def flush(ci):
      @pl.when(c == ci)
      def _():
        off = pl.multiple_of(s * stripe, 128)
        pltpu.sync_copy(acc.at[pl.ds(off, stripe)],
                        out_hbm.at[ci].at[pl.ds(off, stripe)])

    flush(0)
    flush(1)

  return k(dst4, zeros_np)


_G = 8


@functools.partial(jax.jit, static_argnums=(4, 5))
def _sc_aggregate(hws, src3, dst3, zeros_n16, N, E):
  mesh = plsc.VectorSubcoreMesh(core_axis_name="c", subcore_axis_name="s")
  Np = ((N + 127) // 128) * 128
  rpt = Np // 16
  n_rows = src3.shape[1]
  n_groups = n_rows // _G
  n_pairs = n_groups // 2
  assert n_groups % 2 == 0

  @functools.partial(
      pl.kernel,
      out_type=jax.ShapeDtypeStruct((4, Np, 16), jnp.float32),
      mesh=mesh,
      scratch_types=[
          pltpu.VMEM((_G, _CHUNK), jnp.int32),
          pltpu.VMEM((_G, _CHUNK), jnp.int32),
          pltpu.VMEM((_G, _CHUNK), jnp.int32),
          pltpu.VMEM((_G, _CHUNK), jnp.int32),
          pltpu.VMEM((_G, _CHUNK, 16), jnp.float32),
          pltpu.VMEM((_G, _CHUNK, 16), jnp.float32),
          pltpu.VMEM_SHARED((Np, 16), jnp.float32),
          pltpu.SemaphoreType.DMA,
          pltpu.SemaphoreType.DMA,
          pltpu.SemaphoreType.DMA,
          pltpu.SemaphoreType.DMA,
          pltpu.SemaphoreType.DMA,
          pltpu.SemaphoreType.DMA,
      ],
      compiler_params=pltpu.CompilerParams(use_tc_tiling_on_sc=False),
  )
  def k(hws_hbm, src_hbm, dst_hbm, zeros_hbm, out_hbm,
        src_v0, src_v1, dst_v0, dst_v1, rows_v0, rows_v1, acc,
        isem0, isem1, gsem0, gsem1, ssem0, ssem1):
    src_v = [src_v0, src_v1]
    dst_v = [dst_v0, dst_v1]
    rows_v = [rows_v0, rows_v1]
    isem = [isem0, isem1]
    gsem = [gsem0, gsem1]
    ssem = [ssem0, ssem1]
    c = lax.axis_index("c")
    s = lax.axis_index("s")
    row0 = pl.multiple_of(s * rpt, 8)

    def load_idx(buf, g):
      g0 = pl.multiple_of(g * _G, _G)
      ia = pltpu.async_copy(src_hbm.at[s].at[pl.ds(g0, _G)],
                            src_v[buf], isem[buf])
      ib = pltpu.async_copy(dst_hbm.at[s].at[pl.ds(g0, _G)],
                            dst_v[buf], isem[buf])
      return ia, ib

    def fire_gathers(buf, tbl):
      def gbody(b, carry):
        pltpu.async_copy(tbl.at[src_v[buf].at[b]],
                         rows_v[buf].at[b], gsem[buf])
        return carry
      lax.fori_loop(0, _G, gbody, 0)

    def drain_gathers(buf):
      for _ in range(_G):
        pltpu.make_async_copy(zeros_hbm.at[pl.ds(0, _CHUNK)],
                              rows_v[buf].at[0], gsem[buf]).wait()

    def fire_scatters(buf):
      def sbody(b, carry):
        pltpu.async_copy(rows_v[buf].at[b],
                         acc.at[dst_v[buf].at[b]],
                         ssem[buf], add=True)
        return carry
      lax.fori_loop(0, _G, sbody, 0)

    def drain_scatters(buf):
      for _ in range(_G):
        pltpu.make_async_copy(zeros_hbm.at[pl.ds(0, _CHUNK)],
                              rows_v[buf].at[0], ssem[buf]).wait()

    for ci in range(2):
      @pl.when(c == ci)
      def _():
        for p in range(2):
          q = 2 * ci + p
          tbl = hws_hbm.at[q]

          pltpu.sync_copy(zeros_hbm.at[pl.ds(row0, rpt)],
                          acc.at[pl.ds(row0, rpt)])
          plsc.subcore_barrier()

          def body(j, carry):
            ia0, ib0 = load_idx(0, 2 * j)
            ia1, ib1 = load_idx(1, 2 * j + 1)
            ia0.wait(); ib0.wait()
            fire_gathers(0, tbl)
            ia1.wait(); ib1.wait()
            fire_gathers(1, tbl)
            drain_gathers(0)
            fire_scatters(0)
            drain_gathers(1)
            fire_scatters(1)
            drain_scatters(0)
            drain_scatters(1)
            return carry

          lax.fori_loop(0, n_pairs, body, 0)
          plsc.subcore_barrier()
          pltpu.sync_copy(acc.at[pl.ds(row0, rpt)],
                          out_hbm.at[q].at[pl.ds(row0, rpt)])
          plsc.subcore_barrier()

  return k(hws, src3, dst3, zeros_n16)


def _split16(hws, out_ref):
  for q in range(4):
    out_ref[q, ...] = hws[:, 16 * q:16 * (q + 1)]


def _mm1_body(x_ref, w_ref, deg_ref, out_ref):
  dinv = lax.rsqrt(deg_ref[...])
  hws = jnp.dot(x_ref[...], w_ref[...],
                preferred_element_type=jnp.float32) * dinv
  _split16(hws, out_ref)


def _mm1(x, w, deg2d, N, D):
  grid = (N // _BLK,)
  return pl.pallas_call(
      _mm1_body,
      grid=grid,
      in_specs=[
          pl.BlockSpec((_BLK, D), lambda i: (i, 0)),
          pl.BlockSpec((D, 64), lambda i: (0, 0)),
          pl.BlockSpec((_BLK, 1), lambda i: (i, 0)),
      ],
      out_specs=pl.BlockSpec((4, _BLK, 16), lambda i: (0, i, 0)),
      out_shape=jax.ShapeDtypeStruct((4, N, 16), jnp.float32),
  )(x, w, deg2d)


def _core_h(agg_ref, hws_ref, deg_ref, stats_ref, gamma_ref, beta_ref, n):
  dinv = lax.rsqrt(deg_ref[...])
  core = jnp.concatenate(
      [agg_ref[q] + hws_ref[q] for q in range(4)], axis=1) * dinv
  mu = stats_ref[0:1, :] / n
  var = stats_ref[1:2, :] / n - mu * mu
  h = gamma_ref[...] * (core - mu) * lax.rsqrt(var + 1e-5) + beta_ref[...]
  return jnp.maximum(h, 0.0)


def _stats_body(agg_ref, hws_ref, deg_ref, out_ref):
  i = pl.program_id(0)
  @pl.when(i == 0)
  def _():
    out_ref[...] = jnp.zeros_like(out_ref)
  dinv = lax.rsqrt(deg_ref[...])
  core = jnp.concatenate(
      [agg_ref[q] + hws_ref[q] for q in range(4)], axis=1) * dinv
  out_ref[...] += jnp.stack(
      [jnp.sum(core, axis=0), jnp.sum(core * core, axis=0)], axis=0)


def _stats(agg, hws, deg2d, N):
  grid = (N // _BLK,)
  return pl.pallas_call(
      _stats_body,
      grid=grid,
      in_specs=[
          pl.BlockSpec((4, _BLK, 16), lambda i: (0, i, 0)),
          pl.BlockSpec((4, _BLK, 16), lambda i: (0, i, 0)),
          pl.BlockSpec((_BLK, 1), lambda i: (i, 0)),
      ],
      out_specs=pl.BlockSpec((2, 64), lambda i: (0, 0)),
      out_shape=jax.ShapeDtypeStruct((2, 64), jnp.float32),
  )(agg, hws, deg2d)


def _mm23_body(agg_ref, hws_ref, deg_ref, stats_ref, gamma_ref, beta_ref,
               w_ref, out_ref, *, n):
  h = _core_h(agg_ref, hws_ref, deg_ref, stats_ref, gamma_ref, beta_ref, n)
  dinv = lax.rsqrt(deg_ref[...])
  hws = jnp.dot(h, w_ref[...], preferred_element_type=jnp.float32) * dinv
  _split16(hws, out_ref)


def _mm23(agg, hws, deg2d, stats, gamma, beta, w, N):
  grid = (N // _BLK,)
  return pl.pallas_call(
      functools.partial(_mm23_body, n=float(N)),
      grid=grid,
      in_specs=[
          pl.BlockSpec((4, _BLK, 16), lambda i: (0, i, 0)),
          pl.BlockSpec((4, _BLK, 16), lambda i: (0, i, 0)),
          pl.BlockSpec((_BLK, 1), lambda i: (i, 0)),
          pl.BlockSpec((2, 64), lambda i: (0, 0)),
          pl.BlockSpec((1, 64), lambda i: (0, 0)),
          pl.BlockSpec((1, 64), lambda i: (0, 0)),
          pl.BlockSpec((64, 64), lambda i: (0, 0)),
      ],
      out_specs=pl.BlockSpec((4, _BLK, 16), lambda i: (0, i, 0)),
      out_shape=jax.ShapeDtypeStruct((4, N, 16), jnp.float32),
  )(agg, hws, deg2d, stats, gamma, beta, w)


def _final_body(agg_ref, hws_ref, deg_ref, stats_ref, gamma_ref, beta_ref,
                gctx_ref, w1a_ref, w1b_ref, fb1_ref, w2_ref, fb2_ref,
                dw1_ref, db1_ref, dw2_ref, db2_ref, theta_ref,
                delta_ref, thout_ref, *, n):
  h3 = _core_h(agg_ref, hws_ref, deg_ref, stats_ref, gamma_ref, beta_ref, n)
  gv = jnp.dot(gctx_ref[...], w1b_ref[...],
               preferred_element_type=jnp.float32) + fb1_ref[...]
  t = jnp.maximum(
      jnp.dot(h3, w1a_ref[...], preferred_element_type=jnp.float32) + gv, 0.0)
  hf = h3 + jnp.dot(t, w2_ref[...],
                    preferred_element_type=jnp.float32) + fb2_ref[...]
  d1 = jnp.maximum(
      jnp.dot(hf, dw1_ref[...], preferred_element_type=jnp.float32)
      + db1_ref[...], 0.0)
  dfull = jnp.dot(d1, dw2_ref[...], preferred_element_type=jnp.float32)
  delta = dfull[:, 0:1] + db2_ref[...]
  a = theta_ref[...] + delta
  delta_ref[...] = delta
  thout_ref[...] = jnp.arctan2(jnp.sin(a), jnp.cos(a))


def _final(agg, hws, deg2d, stats, gamma, beta, gctx, w1a, w1b, fb1, w2, fb2,
           dw1, db1, dw2p, db2, theta2d, N):
  grid = (N // _BLK,)
  full = lambda shape: pl.BlockSpec(shape, lambda i: tuple(0 for _ in shape))
  return pl.pallas_call(
      functools.partial(_final_body, n=float(N)),
      grid=grid,
      in_specs=[
          pl.BlockSpec((4, _BLK, 16), lambda i: (0, i, 0)),
          pl.BlockSpec((4, _BLK, 16), lambda i: (0, i, 0)),
          pl.BlockSpec((_BLK, 1), lambda i: (i, 0)),
          full((2, 64)),
          full((1, 64)),
          full((1, 64)),
          full((1, 4)),
          full((64, 64)),
          full((4, 64)),
          full((1, 64)),
          full((64, 64)),
          full((1, 64)),
          full((64, 64)),
          full((1, 64)),
          full((64, 128)),
          full((1, 1)),
          pl.BlockSpec((_BLK, 1), lambda i: (i, 0)),
      ],
      out_specs=[
          pl.BlockSpec((_BLK, 1), lambda i: (i, 0)),
          pl.BlockSpec((_BLK, 1), lambda i: (i, 0)),
      ],
      out_shape=[
          jax.ShapeDtypeStruct((N, 1), jnp.float32),
          jax.ShapeDtypeStruct((N, 1), jnp.float32),
      ],
  )(agg, hws, deg2d, stats, gamma, beta, gctx, w1a, w1b, fb1, w2, fb2,
    dw1, db1, dw2p, db2, theta2d)


def kernel(x, edge_index, global_ctx, theta_t, params):
  N, D = x.shape
  E = edge_index.shape[1]
  src = edge_index[0]
  dst = edge_index[1]
  Np = ((N + 16 * 128 - 1) // (16 * 128)) * (16 * 128)
  zeros_np = jnp.zeros((Np,), jnp.float32)
  zeros_n16 = jnp.zeros((((N + 127) // 128) * 128, 16), jnp.float32)

  quant = _CHUNK * 16 * 2 * _G
  E_pad = ((E + quant - 1) // quant) * quant
  srcp = jnp.concatenate([src, jnp.zeros((E_pad - E,), jnp.int32)])
  dstp = jnp.concatenate([dst, jnp.full((E_pad - E,), N, jnp.int32)])
  src3 = srcp.reshape(16, -1, _CHUNK)
  dst3 = dstp.reshape(16, -1, _CHUNK)
  dst4 = dstp.reshape(32, -1, _CHUNK)

  parts = _sc_degree(dst4, zeros_np, N, E)
  deg2d = (parts[0, :N] + parts[1, :N] + 1.0).reshape(N, 1)

  hws = _mm1(x, params["gcn_W"][0], deg2d, N, D)
  r = lambda v: v.reshape(1, -1)
  for li in range(3):
    agg = _sc_aggregate(hws, src3, dst3, zeros_n16, N, E)
    stats = _stats(agg, hws, deg2d, N)
    gamma = r(params["bn_gamma"][li])
    beta = r(params["bn_beta"][li])
    if li < 2:
      hws = _mm23(agg, hws, deg2d, stats, gamma, beta,
                  params["gcn_W"][li + 1], N)
    else:
      dw2p = jnp.pad(params["dec_W2"], ((0, 0), (0, 127)))
      delta2d, theta2d = _final(
          agg, hws, deg2d, stats, gamma, beta, global_ctx,
          params["fus_W1"][:64], params["fus_W1"][64:], r(params["fus_b1"]),
          params["fus_W2"], r(params["fus_b2"]),
          params["dec_W1"], r(params["dec_b1"]),
          dw2p, params["dec_b2"].reshape(1, 1),
          theta_t.reshape(N, 1), N)
  return delta2d[:, 0], theta2d[:, 0]
